# baseline (speedup 1.0000x reference)
import functools

import jax
import jax.numpy as jnp
from jax import lax
from jax.experimental import pallas as pl
from jax.experimental.pallas import tpu as pltpu
from jax.experimental.pallas import tpu_sc as plsc

T = 2048
D = 1024
F = 2048
E = 8
CAP = 640
BISECT_ITERS = 40

NC = 2
NS = 16
L = 16
NW = NC * NS

GCHUNK = 80
ROWS_PER_GW = CAP // 4
CCH = 32
TOK_PER_W = T // NW

def _routing_body(x_ref, wg_ref, idx_ref, sv_ref, pos_ref):
    x_hi = x_ref[...].astype(jnp.bfloat16)
    g_hi = wg_ref[...].astype(jnp.bfloat16)
    logits = jax.lax.dot_general(
        g_hi, x_hi, (((0,), (1,)), ((), ())),
        preferred_element_type=jnp.float32)
    m = jnp.max(logits, axis=0, keepdims=True)
    ex = jnp.exp(logits - m)
    probs = ex / jnp.sum(ex, axis=0, keepdims=True)

    m1 = jnp.max(probs, axis=0, keepdims=True)
    masked = jnp.where(probs == m1, -1.0, probs)
    m2 = jnp.max(masked, axis=0, keepdims=True)
    comb = jnp.where(probs >= m2, probs / (m1 + m2), 0.0)

    lo0 = jnp.zeros((E, 1), jnp.float32)
    hi0 = jnp.max(comb, axis=1, keepdims=True) + 1.0

    def body(_, carry):
        lo, hi = carry
        mid = 0.5 * (lo + hi)
        cnt = jnp.sum((comb >= mid).astype(jnp.float32), axis=1, keepdims=True)
        pred = cnt >= CAP
        return jnp.where(pred, mid, lo), jnp.where(pred, hi, mid)

    lo, _ = jax.lax.fori_loop(0, BISECT_ITERS, body, (lo0, hi0))
    w = jnp.where(comb >= lo, comb, 0.0)
    sel = (w > 0.0).astype(jnp.float32)

    ii = lax.broadcasted_iota(jnp.int32, (T, T), 0)
    jj = lax.broadcasted_iota(jnp.int32, (T, T), 1)
    tri = (ii <= jj).astype(jnp.bfloat16)
    rank = jax.lax.dot_general(
        sel.astype(jnp.bfloat16), tri, (((1,), (0,)), ((), ())),
        preferred_element_type=jnp.float32)
    seli = sel * (rank <= CAP)

    tvec = lax.broadcasted_iota(jnp.int32, (1, T), 1).astype(jnp.float32)
    t_hi = jnp.floor(tvec / 64.0)
    t_lo = tvec - 64.0 * t_hi
    w_h = w.astype(jnp.bfloat16).astype(jnp.float32)
    w_l = w - w_h
    cvec = lax.broadcasted_iota(jnp.int32, (CAP, 1), 0).astype(jnp.float32) + 1.0

    for e in range(E):
        re = rank[e:e + 1, :]
        se = seli[e:e + 1, :] > 0.0
        M = jnp.where((cvec == re) & se, 1.0, 0.0).astype(jnp.bfloat16)
        rhsT = jnp.concatenate(
            [t_hi, t_lo, w_h[e:e + 1, :], w_l[e:e + 1, :]],
            axis=0).astype(jnp.bfloat16)
        outm = jax.lax.dot_general(
            M, rhsT, (((1,), (1,)), ((), ())),
            preferred_element_type=jnp.float32)
        si = 64.0 * outm[:, 0] + outm[:, 1]
        idx_ref[e, :] = si.astype(jnp.int32)
        sv_ref[e, :] = outm[:, 2] + outm[:, 3]

    ecol = lax.broadcasted_iota(jnp.int32, (E, 1), 0).astype(jnp.float32)
    n = rank[:, T - 1:T]
    nmin = jnp.min(n, axis=0, keepdims=True)
    ew = jnp.where(n == nmin, ecol, jnp.float32(E))
    estar = jnp.min(ew, axis=0, keepdims=True)
    zrow = estar * CAP + (CAP - 1.0)
    colpos = jnp.where(seli > 0.0, ecol * CAP + rank - 1.0, zrow)
    is1 = (probs == m1).astype(jnp.float32)
    is2 = ((probs >= m2) & (probs != m1)).astype(jnp.float32)
    pos0 = jnp.sum(is1 * colpos, axis=0, keepdims=True)
    pos1 = jnp.sum(is2 * colpos, axis=0, keepdims=True)
    pos_ref[...] = jnp.concatenate([pos0, pos1], axis=0).astype(jnp.int32)


def _gather_body(x_hbm, idxf_hbm, xe_hbm, ibuf, rbuf, sem):
    wid = lax.axis_index("s") * NC + lax.axis_index("c")
    e = wid // 4
    base = (wid % 4) * ROWS_PER_GW
    for c in range(ROWS_PER_GW // GCHUNK):
        r0 = base + c * GCHUNK
        pltpu.sync_copy(idxf_hbm.at[pl.ds(e * CAP + r0, GCHUNK)], ibuf)
        pltpu.async_copy(x_hbm.at[ibuf], rbuf, sem).wait()
        pltpu.sync_copy(rbuf, xe_hbm.at[e, pl.ds(r0, GCHUNK)])


def _ffn_body(sv_ref, xe_ref, w1_ref, w2_ref, oe_ref):
    wcol = sv_ref[0]
    xb = xe_ref[0].astype(jnp.bfloat16)
    w1 = w1_ref[0].astype(jnp.bfloat16)
    h = jnp.dot(xb, w1, preferred_element_type=jnp.float32)
    h = jnp.maximum(h, 0.0) * wcol
    w2 = w2_ref[0].astype(jnp.bfloat16)
    oe_ref[0] = jnp.dot(h.astype(jnp.bfloat16), w2,
                        preferred_element_type=jnp.float32)


def _combine_body(oe_hbm, posf_hbm, out_hbm, p0, p1, g0, g1, sem0, sem1):
    wid = lax.axis_index("s") * NC + lax.axis_index("c")
    base = wid * TOK_PER_W
    for c in range(TOK_PER_W // CCH):
        tb = base + c * CCH
        pltpu.sync_copy(posf_hbm.at[pl.ds(tb, CCH)], p0)
        pltpu.sync_copy(posf_hbm.at[pl.ds(T + tb, CCH)], p1)
        cp0 = pltpu.async_copy(oe_hbm.at[p0], g0, sem0)
        cp1 = pltpu.async_copy(oe_hbm.at[p1], g1, sem1)
        cp0.wait()
        cp1.wait()

        def arow(j, _):
            for k in range(D // L):
                g0[j, pl.ds(k * L, L)] = (g0[j, pl.ds(k * L, L)]
                                          + g1[j, pl.ds(k * L, L)])
            return 0

        lax.fori_loop(0, CCH, arow, 0)
        pltpu.sync_copy(g0, out_hbm.at[pl.ds(tb, CCH)])


@functools.lru_cache(maxsize=None)
def _sc_kernels():
    mesh = plsc.VectorSubcoreMesh(core_axis_name="c", subcore_axis_name="s",
                                  num_cores=NC, num_subcores=NS)
    gather = pl.kernel(
        _gather_body,
        out_type=jax.ShapeDtypeStruct((E, CAP, D), jnp.float32),
        mesh=mesh,
        scratch_types=[pltpu.VMEM((GCHUNK,), jnp.int32),
                       pltpu.VMEM((GCHUNK, D), jnp.float32),
                       pltpu.SemaphoreType.DMA],
    )
    combine = pl.kernel(
        _combine_body,
        out_type=jax.ShapeDtypeStruct((T, D), jnp.float32),
        mesh=mesh,
        scratch_types=[pltpu.VMEM((CCH,), jnp.int32),
                       pltpu.VMEM((CCH,), jnp.int32),
                       pltpu.VMEM((CCH, D), jnp.float32),
                       pltpu.VMEM((CCH, D), jnp.float32),
                       pltpu.SemaphoreType.DMA,
                       pltpu.SemaphoreType.DMA],
    )
    return gather, combine


def kernel(input, Wg, W1, W2):
    x = input
    idx, sv, pos = pl.pallas_call(
        _routing_body,
        out_shape=(jax.ShapeDtypeStruct((E, CAP), jnp.int32),
                   jax.ShapeDtypeStruct((E, CAP), jnp.float32),
                   jax.ShapeDtypeStruct((2, T), jnp.int32)),
    )(x, Wg)

    _gather, _combine = _sc_kernels()
    xe = _gather(x, idx.reshape(E * CAP))

    oe = pl.pallas_call(
        _ffn_body,
        grid=(E,),
        in_specs=[
            pl.BlockSpec((1, CAP, 1), lambda e: (e, 0, 0)),
            pl.BlockSpec((1, CAP, D), lambda e: (e, 0, 0)),
            pl.BlockSpec((1, D, F), lambda e: (e, 0, 0)),
            pl.BlockSpec((1, F, D), lambda e: (e, 0, 0)),
        ],
        out_specs=pl.BlockSpec((1, CAP, D), lambda e: (e, 0, 0)),
        out_shape=jax.ShapeDtypeStruct((E, CAP, D), jnp.float32),
    )(sv.reshape(E, CAP, 1), xe, W1, W2)

    out = _combine(oe.reshape(E * CAP, D), pos.reshape(2 * T))
    return out


# speedup vs baseline: 1.1684x; 1.1684x over previous
import functools

import jax
import jax.numpy as jnp
from jax import lax
from jax.experimental import pallas as pl
from jax.experimental.pallas import tpu as pltpu
from jax.experimental.pallas import tpu_sc as plsc

T = 2048
D = 1024
F = 2048
E = 8
CAP = 640
BISECT_ITERS = 40

NC = 2
NS = 16
L = 16
NW = NC * NS

FBLK = 1024
NFB = F // FBLK
CCH = 32
TOK_PER_W = T // NW


def _routing_body(x_ref, wg_ref, rank_ref, w_ref, pos_ref):
    x_hi = x_ref[...].astype(jnp.bfloat16)
    g_hi = wg_ref[...].astype(jnp.bfloat16)
    logits = jax.lax.dot_general(
        g_hi, x_hi, (((0,), (1,)), ((), ())),
        preferred_element_type=jnp.float32)
    m = jnp.max(logits, axis=0, keepdims=True)
    ex = jnp.exp(logits - m)
    probs = ex / jnp.sum(ex, axis=0, keepdims=True)

    m1 = jnp.max(probs, axis=0, keepdims=True)
    masked = jnp.where(probs == m1, -1.0, probs)
    m2 = jnp.max(masked, axis=0, keepdims=True)
    comb = jnp.where(probs >= m2, probs / (m1 + m2), 0.0)

    lo0 = jnp.zeros((E, 1), jnp.float32)
    hi0 = jnp.max(comb, axis=1, keepdims=True) + 1.0

    def body(_, carry):
        lo, hi = carry
        mid = 0.5 * (lo + hi)
        cnt = jnp.sum((comb >= mid).astype(jnp.float32), axis=1, keepdims=True)
        pred = cnt >= CAP
        return jnp.where(pred, mid, lo), jnp.where(pred, hi, mid)

    lo, _ = jax.lax.fori_loop(0, BISECT_ITERS, body, (lo0, hi0))
    w = jnp.where(comb >= lo, comb, 0.0)
    sel = (w > 0.0).astype(jnp.float32)

    ii = lax.broadcasted_iota(jnp.int32, (T, T), 0)
    jj = lax.broadcasted_iota(jnp.int32, (T, T), 1)
    tri = (ii <= jj).astype(jnp.bfloat16)
    rank = jax.lax.dot_general(
        sel.astype(jnp.bfloat16), tri, (((1,), (0,)), ((), ())),
        preferred_element_type=jnp.float32)
    seli = sel * (rank <= CAP)

    rank_ref[...] = rank
    w_ref[...] = jnp.where(seli > 0.0, w, 0.0)

    ecol = lax.broadcasted_iota(jnp.int32, (E, 1), 0).astype(jnp.float32)
    n = rank[:, T - 1:T]
    nmin = jnp.min(n, axis=0, keepdims=True)
    ew = jnp.where(n == nmin, ecol, jnp.float32(E))
    estar = jnp.min(ew, axis=0, keepdims=True)
    zrow = estar * CAP + (CAP - 1.0)
    colpos = jnp.where(seli > 0.0, ecol * CAP + rank - 1.0, zrow)
    is1 = (probs == m1).astype(jnp.float32)
    is2 = ((probs >= m2) & (probs != m1)).astype(jnp.float32)
    pos0 = jnp.sum(is1 * colpos, axis=0, keepdims=True)
    pos1 = jnp.sum(is2 * colpos, axis=0, keepdims=True)
    pos_ref[...] = jnp.concatenate([pos0, pos1], axis=0).astype(jnp.int32)


def _ffn_body(rank_ref, w_ref, x_ref, w1_ref, w2_ref, oe_ref):
    e = pl.program_id(0)
    fb = pl.program_id(1)

    ohe = (lax.broadcasted_iota(jnp.int32, (1, E), 1) == e).astype(jnp.float32)
    re = jnp.dot(ohe, rank_ref[...], preferred_element_type=jnp.float32)
    we = jnp.dot(ohe, w_ref[...], preferred_element_type=jnp.float32)

    cvec = (lax.broadcasted_iota(jnp.int32, (CAP, 1), 0)
            .astype(jnp.float32) + 1.0)
    M = jnp.where((cvec == re) & (we > 0.0), 1.0, 0.0).astype(jnp.bfloat16)

    xe = jnp.dot(M, x_ref[...].astype(jnp.bfloat16),
                 preferred_element_type=jnp.float32)
    w_h = we.astype(jnp.bfloat16).astype(jnp.float32)
    w_l = we - w_h
    wcat = jnp.concatenate([w_h, w_l], axis=0).astype(jnp.bfloat16)
    svT = jax.lax.dot_general(M, wcat, (((1,), (1,)), ((), ())),
                              preferred_element_type=jnp.float32)
    wcol = svT[:, 0:1] + svT[:, 1:2]

    h = jnp.dot(xe.astype(jnp.bfloat16), w1_ref[0].astype(jnp.bfloat16),
                preferred_element_type=jnp.float32)
    h = jnp.maximum(h, 0.0) * wcol
    part = jnp.dot(h.astype(jnp.bfloat16), w2_ref[0].astype(jnp.bfloat16),
                   preferred_element_type=jnp.float32)

    @pl.when(fb == 0)
    def _():
        oe_ref[0] = part

    @pl.when(fb != 0)
    def _():
        oe_ref[0] += part


def _combine_body(oe_hbm, posf_hbm, out_hbm, p0, p1, g0, g1, sem0, sem1):
    wid = lax.axis_index("s") * NC + lax.axis_index("c")
    base = wid * TOK_PER_W
    for c in range(TOK_PER_W // CCH):
        tb = base + c * CCH
        pltpu.sync_copy(posf_hbm.at[pl.ds(tb, CCH)], p0)
        pltpu.sync_copy(posf_hbm.at[pl.ds(T + tb, CCH)], p1)
        cp0 = pltpu.async_copy(oe_hbm.at[p0], g0, sem0)
        cp1 = pltpu.async_copy(oe_hbm.at[p1], g1, sem1)
        cp0.wait()
        cp1.wait()

        def arow(j, _):
            for k in range(D // L):
                g0[j, pl.ds(k * L, L)] = (g0[j, pl.ds(k * L, L)]
                                          + g1[j, pl.ds(k * L, L)])
            return 0

        lax.fori_loop(0, CCH, arow, 0)
        pltpu.sync_copy(g0, out_hbm.at[pl.ds(tb, CCH)])


@functools.lru_cache(maxsize=None)
def _sc_kernels():
    mesh = plsc.VectorSubcoreMesh(core_axis_name="c", subcore_axis_name="s",
                                  num_cores=NC, num_subcores=NS)
    combine = pl.kernel(
        _combine_body,
        out_type=jax.ShapeDtypeStruct((T, D), jnp.float32),
        mesh=mesh,
        scratch_types=[pltpu.VMEM((CCH,), jnp.int32),
                       pltpu.VMEM((CCH,), jnp.int32),
                       pltpu.VMEM((CCH, D), jnp.float32),
                       pltpu.VMEM((CCH, D), jnp.float32),
                       pltpu.SemaphoreType.DMA,
                       pltpu.SemaphoreType.DMA],
    )
    return combine


def kernel(input, Wg, W1, W2):
    x = input
    rank, w, pos = pl.pallas_call(
        _routing_body,
        out_shape=(jax.ShapeDtypeStruct((E, T), jnp.float32),
                   jax.ShapeDtypeStruct((E, T), jnp.float32),
                   jax.ShapeDtypeStruct((2, T), jnp.int32)),
    )(x, Wg)

    oe = pl.pallas_call(
        _ffn_body,
        grid=(E, NFB),
        in_specs=[
            pl.BlockSpec((E, T), lambda e, fb: (0, 0)),
            pl.BlockSpec((E, T), lambda e, fb: (0, 0)),
            pl.BlockSpec((T, D), lambda e, fb: (0, 0)),
            pl.BlockSpec((1, D, FBLK), lambda e, fb: (e, 0, fb)),
            pl.BlockSpec((1, FBLK, D), lambda e, fb: (e, fb, 0)),
        ],
        out_specs=pl.BlockSpec((1, CAP, D), lambda e, fb: (e, 0, 0)),
        out_shape=jax.ShapeDtypeStruct((E, CAP, D), jnp.float32),
    )(rank, w, x, W1, W2)

    combine = _sc_kernels()
    out = combine(oe.reshape(E * CAP, D), pos.reshape(2 * T))
    return out


# speedup vs baseline: 1.1798x; 1.0098x over previous
import functools

import jax
import jax.numpy as jnp
from jax import lax
from jax.experimental import pallas as pl
from jax.experimental.pallas import tpu as pltpu
from jax.experimental.pallas import tpu_sc as plsc

T = 2048
D = 1024
F = 2048
E = 8
CAP = 640
BISECT_ITERS = 40

NC = 2
NS = 16
L = 16
NW = NC * NS

FBLK = 1024
NFB = F // FBLK
CCH = 32
TOK_PER_W = T // NW


def _routing_body(x_ref, wg_ref, rank_ref, w_ref, pos_ref):
    x_hi = x_ref[...].astype(jnp.bfloat16)
    g_hi = wg_ref[...].astype(jnp.bfloat16)
    logits = jax.lax.dot_general(
        g_hi, x_hi, (((0,), (1,)), ((), ())),
        preferred_element_type=jnp.float32)
    m = jnp.max(logits, axis=0, keepdims=True)
    ex = jnp.exp(logits - m)
    probs = ex / jnp.sum(ex, axis=0, keepdims=True)

    m1 = jnp.max(probs, axis=0, keepdims=True)
    masked = jnp.where(probs == m1, -1.0, probs)
    m2 = jnp.max(masked, axis=0, keepdims=True)
    comb = jnp.where(probs >= m2, probs / (m1 + m2), 0.0)

    lo0 = jnp.zeros((E, 1), jnp.float32)
    hi0 = jnp.max(comb, axis=1, keepdims=True) + 1.0

    def body(_, carry):
        lo, hi = carry
        mid = 0.5 * (lo + hi)
        cnt = jnp.sum((comb >= mid).astype(jnp.float32), axis=1, keepdims=True)
        pred = cnt >= CAP
        return jnp.where(pred, mid, lo), jnp.where(pred, hi, mid)

    lo, _ = jax.lax.fori_loop(0, BISECT_ITERS, body, (lo0, hi0))
    w = jnp.where(comb >= lo, comb, 0.0)
    sel = (w > 0.0).astype(jnp.float32)

    ii = lax.broadcasted_iota(jnp.int32, (T, T), 0)
    jj = lax.broadcasted_iota(jnp.int32, (T, T), 1)
    tri = (ii <= jj).astype(jnp.bfloat16)
    rank = jax.lax.dot_general(
        sel.astype(jnp.bfloat16), tri, (((1,), (0,)), ((), ())),
        preferred_element_type=jnp.float32)
    seli = sel * (rank <= CAP)

    rank_ref[...] = rank
    w_ref[...] = jnp.where(seli > 0.0, w, 0.0)

    ecol = lax.broadcasted_iota(jnp.int32, (E, 1), 0).astype(jnp.float32)
    n = rank[:, T - 1:T]
    nmin = jnp.min(n, axis=0, keepdims=True)
    ew = jnp.where(n == nmin, ecol, jnp.float32(E))
    estar = jnp.min(ew, axis=0, keepdims=True)
    zrow = estar * CAP + (CAP - 1.0)
    colpos = jnp.where(seli > 0.0, ecol * CAP + rank - 1.0, zrow)
    is1 = (probs == m1).astype(jnp.float32)
    is2 = ((probs >= m2) & (probs != m1)).astype(jnp.float32)
    pos0 = jnp.sum(is1 * colpos, axis=0, keepdims=True)
    pos1 = jnp.sum(is2 * colpos, axis=0, keepdims=True)
    pos_ref[...] = jnp.concatenate([pos0, pos1], axis=0).astype(jnp.int32)


def _ffn_body(rank_ref, w_ref, x_ref, w1_ref, w2_ref, oe_ref):
    e = pl.program_id(0)
    fb = pl.program_id(1)

    ohe = (lax.broadcasted_iota(jnp.int32, (1, E), 1) == e).astype(jnp.bfloat16)
    rk = rank_ref[...]
    rk_hi = jnp.floor(rk * (1.0 / 64.0))
    rk_lo = rk - 64.0 * rk_hi

    def pick(row_f32):
        return jnp.dot(ohe, row_f32.astype(jnp.bfloat16),
                       preferred_element_type=jnp.float32)

    re = 64.0 * pick(rk_hi) + pick(rk_lo)
    wv = w_ref[...]
    wv_hi = wv.astype(jnp.bfloat16).astype(jnp.float32)
    we = pick(wv_hi) + pick(wv - wv_hi)

    cvec = (lax.broadcasted_iota(jnp.int32, (CAP, 1), 0)
            .astype(jnp.float32) + 1.0)
    M = jnp.where((cvec == re) & (we > 0.0), 1.0, 0.0).astype(jnp.bfloat16)

    xe = jnp.dot(M, x_ref[...].astype(jnp.bfloat16),
                 preferred_element_type=jnp.float32)
    w_h = we.astype(jnp.bfloat16).astype(jnp.float32)
    w_l = we - w_h
    wcat = jnp.concatenate([w_h, w_l], axis=0).astype(jnp.bfloat16)
    svT = jax.lax.dot_general(M, wcat, (((1,), (1,)), ((), ())),
                              preferred_element_type=jnp.float32)
    wcol = svT[:, 0:1] + svT[:, 1:2]

    h = jnp.dot(xe.astype(jnp.bfloat16), w1_ref[0].astype(jnp.bfloat16),
                preferred_element_type=jnp.float32)
    h = jnp.maximum(h, 0.0) * wcol
    part = jnp.dot(h.astype(jnp.bfloat16), w2_ref[0].astype(jnp.bfloat16),
                   preferred_element_type=jnp.float32)

    @pl.when(fb == 0)
    def _():
        oe_ref[0] = part

    @pl.when(fb != 0)
    def _():
        oe_ref[0] += part


def _combine_body(oe_hbm, posf_hbm, out_hbm, p0, p1, g0, g1, sem0, sem1):
    wid = lax.axis_index("s") * NC + lax.axis_index("c")
    base = wid * TOK_PER_W
    for c in range(TOK_PER_W // CCH):
        tb = base + c * CCH
        pltpu.sync_copy(posf_hbm.at[pl.ds(tb, CCH)], p0)
        pltpu.sync_copy(posf_hbm.at[pl.ds(T + tb, CCH)], p1)
        cp0 = pltpu.async_copy(oe_hbm.at[p0], g0, sem0)
        cp1 = pltpu.async_copy(oe_hbm.at[p1], g1, sem1)
        cp0.wait()
        cp1.wait()

        def arow(j, _):
            for k in range(D // L):
                g0[j, pl.ds(k * L, L)] = (g0[j, pl.ds(k * L, L)]
                                          + g1[j, pl.ds(k * L, L)])
            return 0

        lax.fori_loop(0, CCH, arow, 0)
        pltpu.sync_copy(g0, out_hbm.at[pl.ds(tb, CCH)])


@functools.lru_cache(maxsize=None)
def _sc_kernels():
    mesh = plsc.VectorSubcoreMesh(core_axis_name="c", subcore_axis_name="s",
                                  num_cores=NC, num_subcores=NS)
    combine = pl.kernel(
        _combine_body,
        out_type=jax.ShapeDtypeStruct((T, D), jnp.float32),
        mesh=mesh,
        scratch_types=[pltpu.VMEM((CCH,), jnp.int32),
                       pltpu.VMEM((CCH,), jnp.int32),
                       pltpu.VMEM((CCH, D), jnp.float32),
                       pltpu.VMEM((CCH, D), jnp.float32),
                       pltpu.SemaphoreType.DMA,
                       pltpu.SemaphoreType.DMA],
    )
    return combine


def kernel(input, Wg, W1, W2):
    x = input
    rank, w, pos = pl.pallas_call(
        _routing_body,
        out_shape=(jax.ShapeDtypeStruct((E, T), jnp.float32),
                   jax.ShapeDtypeStruct((E, T), jnp.float32),
                   jax.ShapeDtypeStruct((2, T), jnp.int32)),
    )(x, Wg)

    oe = pl.pallas_call(
        _ffn_body,
        grid=(E, NFB),
        in_specs=[
            pl.BlockSpec((E, T), lambda e, fb: (0, 0)),
            pl.BlockSpec((E, T), lambda e, fb: (0, 0)),
            pl.BlockSpec((T, D), lambda e, fb: (0, 0)),
            pl.BlockSpec((1, D, FBLK), lambda e, fb: (e, 0, fb)),
            pl.BlockSpec((1, FBLK, D), lambda e, fb: (e, fb, 0)),
        ],
        out_specs=pl.BlockSpec((1, CAP, D), lambda e, fb: (e, 0, 0)),
        out_shape=jax.ShapeDtypeStruct((E, CAP, D), jnp.float32),
    )(rank, w, x, W1, W2)

    combine = _sc_kernels()
    out = combine(oe.reshape(E * CAP, D), pos.reshape(2 * T))
    return out
